# baseline (speedup 1.0000x reference)
import jax
import jax.numpy as jnp
from jax import lax
from jax.experimental import pallas as pl
from jax.experimental.pallas import tpu as pltpu

_KA = (1832780943, 270669613)
_KB2 = (2350016172, 1168365246)
_KC = (2465931498, 255383827)

_NUM_T = 16
_NUM_SCALES = 31
_G = 33


def _rotl(x, d):
    return (x << jnp.uint32(d)) | (x >> jnp.uint32(32 - d))


def _threefry_bits(ks0, ks1, cnt):
    ks2 = ks0 ^ ks1 ^ jnp.uint32(0x1BD11BDA)
    ks = (ks0, ks1, ks2)
    rot_a = (13, 15, 26, 6)
    rot_b = (17, 29, 16, 24)
    x0 = jnp.zeros_like(cnt) + ks0
    x1 = cnt + ks1
    for i in range(5):
        for r in (rot_a if i % 2 == 0 else rot_b):
            x0 = x0 + x1
            x1 = _rotl(x1, r)
            x1 = x1 ^ x0
        x0 = x0 + ks[(i + 1) % 3]
        x1 = x1 + ks[(i + 2) % 3] + jnp.uint32(i + 1)
    return x0 ^ x1


def _body(op_ref, nte_ref, se_ref, q_ref, pnst_ref, aug_ref, sc_ref, lp_ref):
    rows = aug_ref.shape[0]
    base = (pl.program_id(0) * rows).astype(jnp.uint32)

    lane = lax.broadcasted_iota(jnp.uint32, (rows, 128), 1)
    row = lax.broadcasted_iota(jnp.uint32, (rows, 128), 0) + base
    in_g0 = lane < _G
    in_g1 = lane < 2 * _G
    is_sc = lane < 3 * _G
    is_nm = jnp.logical_and(lane >= 99, lane < 103)
    slot = jnp.where(in_g0, jnp.uint32(0), jnp.where(in_g1, jnp.uint32(1), jnp.uint32(2)))
    c0 = lane - slot * jnp.uint32(2)
    cnt = jnp.where(is_sc, row * jnp.uint32(93) + c0,
                    jnp.where(is_nm, row * jnp.uint32(4) + (lane - jnp.uint32(99)),
                              row * jnp.uint32(3) + (lane - jnp.uint32(103))))
    k0 = jnp.where(is_sc, jnp.uint32(_KC[0]),
                   jnp.where(is_nm, jnp.uint32(_KA[0]), jnp.uint32(_KB2[0])))
    k1 = jnp.where(is_sc, jnp.uint32(_KC[1]),
                   jnp.where(is_nm, jnp.uint32(_KA[1]), jnp.uint32(_KB2[1])))

    bits = _threefry_bits(k0, k1, cnt)

    tiny = jnp.float32(1.1754943508222875e-38)
    f = lax.bitcast_convert_type((bits >> jnp.uint32(9)) | jnp.uint32(0x3F800000),
                                 jnp.float32) - jnp.float32(1.0)
    u = jnp.maximum(tiny, f * (jnp.float32(1.0) - tiny) + tiny)
    g = -jnp.log(-jnp.log(u))

    g_num = g[:, 99:103]
    ibits = bits[:, 103:106]

    nl = lax.dot_general(q_ref[...], nte_ref[...], (((1,), (1,)), ((), ())),
                         preferred_element_type=jnp.float32)
    nm = jnp.max(nl, axis=1, keepdims=True)
    nlms = nl - nm - jnp.log(jnp.sum(jnp.exp(nl - nm), axis=1, keepdims=True))

    z = nl + g_num
    zm = jnp.max(z, axis=1, keepdims=True)
    i4 = lax.broadcasted_iota(jnp.int32, (rows, 4), 1)
    sidx = jnp.min(jnp.where(z == zm, i4, 4), axis=1, keepdims=True)

    pn = pnst_ref[...]
    n_t = jnp.sum(jnp.where(i4 == sidx, pn, 0), axis=1, keepdims=True)
    lp_total = jnp.sum(jnp.where(i4 == sidx, nlms, jnp.float32(0.0)),
                       axis=1, keepdims=True)

    tl = lax.dot_general(op_ref[...], se_ref[...], (((1,), (1,)), ((), ())),
                         preferred_element_type=jnp.float32)
    tm = jnp.max(tl, axis=1, keepdims=True)
    lse = jnp.log(jnp.sum(jnp.exp(tl - tm), axis=1, keepdims=True))
    tbl = jnp.concatenate(
        [tl, tm, lse, tl, tm, lse, tl, tm, lse, jnp.zeros((_NUM_T, 29), jnp.float32)],
        axis=1)

    augs = []
    for l in range(3):
        raw = (ibits[:, l:l + 1] & jnp.uint32(15)).astype(jnp.int32)
        augs.append(jnp.where(n_t <= l, 0, raw))
    aug_all = jnp.where(in_g0, augs[0], jnp.where(in_g1, augs[1], augs[2]))

    r = jnp.broadcast_to(tbl[0:1, :], (rows, 128))
    for i in range(1, _NUM_T):
        r = jnp.where(aug_all == i, tbl[i:i + 1, :], r)

    zz = r + g
    i31 = lax.broadcasted_iota(jnp.int32, (rows, _NUM_SCALES), 1)
    scs = []
    for l in range(3):
        lo = l * _G
        zsl = zz[:, lo:lo + _NUM_SCALES]
        zsm = jnp.max(zsl, axis=1, keepdims=True)
        sc = jnp.min(jnp.where(zsl == zsm, i31, _NUM_SCALES),
                     axis=1, keepdims=True)
        picked = jnp.sum(jnp.where(i31 == sc, r[:, lo:lo + _NUM_SCALES],
                                   jnp.float32(0.0)), axis=1, keepdims=True)
        lp = (picked - r[:, lo + 31:lo + 32]) - r[:, lo + 32:lo + 33]
        lp_total = lp_total + jnp.where(n_t <= l, jnp.float32(0.0), lp)
        scs.append(sc)
    aug_ref[...] = jnp.concatenate(augs, axis=1)
    sc_ref[...] = jnp.concatenate(scs, axis=1)
    lp_ref[...] = lp_total


def kernel(imgs, op_embs, num_transforms_embs, scale_embs, q, pnst):
    b = imgs.shape[0]
    hidden = q.shape[0]
    rows = 2048
    grid = b // rows
    q2 = q.reshape(1, hidden)
    pn2 = pnst.reshape(1, 4).astype(jnp.int32)

    full = lambda shape: pl.BlockSpec(shape, lambda i: (0,) * len(shape))

    aug, scales, logps = pl.pallas_call(
        _body,
        grid=(grid,),
        in_specs=[full(op_embs.shape), full(num_transforms_embs.shape),
                  full(scale_embs.shape), full(q2.shape), full(pn2.shape)],
        out_specs=[pl.BlockSpec((rows, 3), lambda i: (i, 0)),
                   pl.BlockSpec((rows, 3), lambda i: (i, 0)),
                   pl.BlockSpec((rows, 1), lambda i: (i, 0))],
        out_shape=[jax.ShapeDtypeStruct((b, 3), jnp.int32),
                   jax.ShapeDtypeStruct((b, 3), jnp.int32),
                   jax.ShapeDtypeStruct((b, 1), jnp.float32)],
        compiler_params=pltpu.CompilerParams(
            dimension_semantics=("arbitrary",)),
    )(op_embs, num_transforms_embs, scale_embs, q2, pn2)

    return aug, scales, logps[:, 0]


# speedup vs baseline: 1.0500x; 1.0500x over previous
import jax
import jax.numpy as jnp
from jax import lax
from jax.experimental import pallas as pl
from jax.experimental.pallas import tpu as pltpu

_KA = (1832780943, 270669613)
_KB2 = (2350016172, 1168365246)
_KC = (2465931498, 255383827)

_NUM_T = 16
_NUM_SCALES = 31
_G = 33


def _rotl(x, d):
    return (x << jnp.uint32(d)) | (x >> jnp.uint32(32 - d))


def _threefry_bits(ks0, ks1, cnt):
    ks2 = ks0 ^ ks1 ^ jnp.uint32(0x1BD11BDA)
    ks = (ks0, ks1, ks2)
    rot_a = (13, 15, 26, 6)
    rot_b = (17, 29, 16, 24)
    x0 = jnp.zeros_like(cnt) + ks0
    x1 = cnt + ks1
    for i in range(5):
        for r in (rot_a if i % 2 == 0 else rot_b):
            x0 = x0 + x1
            x1 = _rotl(x1, r)
            x1 = x1 ^ x0
        x0 = x0 + ks[(i + 1) % 3]
        x1 = x1 + ks[(i + 2) % 3] + jnp.uint32(i + 1)
    return x0 ^ x1


def _body(op_ref, nte_ref, se_ref, q_ref, pnst_ref, aug_ref, sc_ref, lp_ref):
    rows = aug_ref.shape[0]
    base = (pl.program_id(0) * rows).astype(jnp.uint32)

    lane = lax.broadcasted_iota(jnp.uint32, (1, 128), 1)
    in_g0 = lane < _G
    in_g1 = lane < 2 * _G
    is_sc = lane < 3 * _G
    is_nm = jnp.logical_and(lane >= 99, lane < 103)
    slot = jnp.where(in_g0, jnp.uint32(0), jnp.where(in_g1, jnp.uint32(1), jnp.uint32(2)))
    c0 = lane - slot * jnp.uint32(2)
    mult = jnp.where(is_sc, jnp.uint32(93),
                     jnp.where(is_nm, jnp.uint32(4), jnp.uint32(3)))
    off = jnp.where(is_sc, c0,
                    jnp.where(is_nm, lane - jnp.uint32(99), lane - jnp.uint32(103)))
    k0 = jnp.where(is_sc, jnp.uint32(_KC[0]),
                   jnp.where(is_nm, jnp.uint32(_KA[0]), jnp.uint32(_KB2[0])))
    k1 = jnp.where(is_sc, jnp.uint32(_KC[1]),
                   jnp.where(is_nm, jnp.uint32(_KA[1]), jnp.uint32(_KB2[1])))

    row = lax.broadcasted_iota(jnp.uint32, (rows, 128), 0) + base
    cnt = row * mult + off

    bits = _threefry_bits(k0, k1, cnt)

    tiny = jnp.float32(1.1754943508222875e-38)
    f = lax.bitcast_convert_type((bits >> jnp.uint32(9)) | jnp.uint32(0x3F800000),
                                 jnp.float32) - jnp.float32(1.0)
    u = jnp.maximum(tiny, f * (jnp.float32(1.0) - tiny) + tiny)
    g = -jnp.log(-jnp.log(u))

    g_num = g[:, 99:103]
    ibits = bits[:, 103:106]

    nl = lax.dot_general(q_ref[...], nte_ref[...], (((1,), (1,)), ((), ())),
                         preferred_element_type=jnp.float32)
    nm = jnp.max(nl, axis=1, keepdims=True)
    nlms = nl - nm - jnp.log(jnp.sum(jnp.exp(nl - nm), axis=1, keepdims=True))

    z = nl + g_num
    zm = jnp.max(z, axis=1, keepdims=True)
    i4 = lax.broadcasted_iota(jnp.int32, (1, 4), 1)
    sidx = jnp.min(jnp.where(z == zm, i4, 4), axis=1, keepdims=True)

    pn = pnst_ref[...]
    n_t = jnp.sum(jnp.where(i4 == sidx, pn, 0), axis=1, keepdims=True)
    lp_total = jnp.sum(jnp.where(i4 == sidx, nlms, jnp.float32(0.0)),
                       axis=1, keepdims=True)

    tl = lax.dot_general(op_ref[...], se_ref[...], (((1,), (1,)), ((), ())),
                         preferred_element_type=jnp.float32)
    tm = jnp.max(tl, axis=1, keepdims=True)
    lse = jnp.log(jnp.sum(jnp.exp(tl - tm), axis=1, keepdims=True))
    tbl = jnp.concatenate(
        [tl, tm, lse, tl, tm, lse, tl, tm, lse, jnp.zeros((_NUM_T, 29), jnp.float32)],
        axis=1)

    augs = []
    for l in range(3):
        raw = (ibits[:, l:l + 1] & jnp.uint32(15)).astype(jnp.int32)
        augs.append(jnp.where(n_t <= l, 0, raw))
    aug_all = jnp.where(in_g0, augs[0], jnp.where(in_g1, augs[1], augs[2]))

    r = jnp.broadcast_to(tbl[0:1, :], (rows, 128))
    for i in range(1, _NUM_T):
        r = jnp.where(aug_all == i, tbl[i:i + 1, :], r)

    zz = r + g
    i31 = lax.broadcasted_iota(jnp.int32, (1, _NUM_SCALES), 1)
    scs = []
    for l in range(3):
        lo = l * _G
        zsl = zz[:, lo:lo + _NUM_SCALES]
        zsm = jnp.max(zsl, axis=1, keepdims=True)
        sc = jnp.min(jnp.where(zsl == zsm, i31, _NUM_SCALES),
                     axis=1, keepdims=True)
        picked = jnp.sum(jnp.where(i31 == sc, r[:, lo:lo + _NUM_SCALES],
                                   jnp.float32(0.0)), axis=1, keepdims=True)
        lp = (picked - r[:, lo + 31:lo + 32]) - r[:, lo + 32:lo + 33]
        lp_total = lp_total + jnp.where(n_t <= l, jnp.float32(0.0), lp)
        scs.append(sc)
    aug_ref[...] = jnp.concatenate(augs, axis=1)
    sc_ref[...] = jnp.concatenate(scs, axis=1)
    lp_ref[...] = lp_total


def kernel(imgs, op_embs, num_transforms_embs, scale_embs, q, pnst):
    b = imgs.shape[0]
    hidden = q.shape[0]
    rows = 4096
    grid = b // rows
    q2 = q.reshape(1, hidden)
    pn2 = pnst.reshape(1, 4).astype(jnp.int32)

    full = lambda shape: pl.BlockSpec(shape, lambda i: (0,) * len(shape))

    aug, scales, logps = pl.pallas_call(
        _body,
        grid=(grid,),
        in_specs=[full(op_embs.shape), full(num_transforms_embs.shape),
                  full(scale_embs.shape), full(q2.shape), full(pn2.shape)],
        out_specs=[pl.BlockSpec((rows, 3), lambda i: (i, 0)),
                   pl.BlockSpec((rows, 3), lambda i: (i, 0)),
                   pl.BlockSpec((rows, 1), lambda i: (i, 0))],
        out_shape=[jax.ShapeDtypeStruct((b, 3), jnp.int32),
                   jax.ShapeDtypeStruct((b, 3), jnp.int32),
                   jax.ShapeDtypeStruct((b, 1), jnp.float32)],
        compiler_params=pltpu.CompilerParams(
            dimension_semantics=("arbitrary",)),
    )(op_embs, num_transforms_embs, scale_embs, q2, pn2)

    return aug, scales, logps[:, 0]
